# baseline (speedup 1.0000x reference)
import functools

import jax
import jax.numpy as jnp
from jax import lax
from jax.experimental import pallas as pl
from jax.experimental.pallas import tpu as pltpu
from jax.experimental.pallas import tpu_sc as plsc

D = 32
NC = 2
NS = 16
NW = NC * NS
G = 128
K = 4
CH = G * K


def _flat_gather(idx2d, table):
    n_rows, _ = idx2d.shape
    n = n_rows * G
    b_per_w = n // NW
    g_per_w = b_per_w // G
    n_chunks = b_per_w // CH
    mesh = plsc.VectorSubcoreMesh(core_axis_name="c", subcore_axis_name="s")

    @functools.partial(
        pl.kernel,
        mesh=mesh,
        out_type=jax.ShapeDtypeStruct((n, D), jnp.float32),
        scratch_types=[
            pltpu.VMEM((g_per_w, G), jnp.int32),
            pltpu.VMEM((2, CH, D), jnp.float32),
            pltpu.SemaphoreType.DMA,
            pltpu.SemaphoreType.DMA,
        ],
        compiler_params=pltpu.CompilerParams(use_tc_tiling_on_sc=False),
    )
    def k(idx_hbm, table_hbm, out_hbm, idx_v, rows_v, sem_g, sem_o):
        wid = lax.axis_index("s") * NC + lax.axis_index("c")
        base = wid * b_per_w

        pltpu.sync_copy(idx_hbm.at[pl.ds(wid * g_per_w, g_per_w)], idx_v)

        def fire_gather(c, b):
            for j in range(K):
                pltpu.async_copy(
                    table_hbm.at[idx_v.at[c * K + j]],
                    rows_v.at[b].at[pl.ds(j * G, G)],
                    sem_g,
                )

        def drain_gather(b):
            pltpu.make_async_copy(
                table_hbm.at[pl.ds(0, CH)], rows_v.at[b], sem_g
            ).wait()

        def fire_out(c, b):
            pltpu.async_copy(
                rows_v.at[b], out_hbm.at[pl.ds(base + c * CH, CH)], sem_o
            )

        def wait_out(b):
            pltpu.make_async_copy(
                rows_v.at[b], out_hbm.at[pl.ds(base, CH)], sem_o
            ).wait()

        fire_gather(0, 0)
        drain_gather(0)
        fire_gather(1, 1)
        fire_out(0, 0)

        def body(t, carry):
            c = 2 * t + 1
            drain_gather(1)
            wait_out(0)
            fire_gather(c + 1, 0)
            fire_out(c, 1)
            drain_gather(0)
            wait_out(1)
            fire_gather(c + 2, 1)
            fire_out(c + 1, 0)
            return carry

        lax.fori_loop(0, (n_chunks - 2) // 2, body, 0)

        drain_gather(1)
        wait_out(0)
        fire_out(n_chunks - 1, 1)
        wait_out(1)

    return k(idx2d, table)


def kernel(x, table):
    b, f = x.shape
    idx2d = x.reshape(b * f // G, G).astype(jnp.int32)
    out = _flat_gather(idx2d, table)
    return out.reshape(b, f, D)
